# baseline (speedup 1.0000x reference)
import functools

import jax
import jax.numpy as jnp
from jax import lax
from jax.experimental import pallas as pl
from jax.experimental.pallas import tpu as pltpu
from jax.experimental.pallas import tpu_sc as plsc

ALPHA_F, GAMMA_F = 0.25, 1.0
ALPHA_R, GAMMA_R, BETA_R = 0.5, 1.5, 0.11
VAR0, VAR1 = 0.1, 0.2
_B = 2.718281828459045 ** (GAMMA_R / ALPHA_R) - 1.0


def _match_kernel(priors_ref, targets_ref, loc_ref,
                  gidx_ref, posf_ref, ignf_ref,
                  loc_sum_ref, pos_cnt_ref, ign_cnt_ref):
    P = priors_ref.shape[1]
    n_obj = targets_ref.shape[1]
    C = 80
    img = pl.program_id(0)
    big = jnp.int32(2 ** 30)

    pcx = priors_ref[0:1, :]
    pcy = priors_ref[1:2, :]
    pw = priors_ref[2:3, :]
    ph = priors_ref[3:4, :]
    px1 = pcx - pw / 2.0
    py1 = pcy - ph / 2.0
    px2 = pcx + pw / 2.0
    py2 = pcy + ph / 2.0

    tx1 = targets_ref[0, :, 0:1]
    ty1 = targets_ref[0, :, 1:2]
    tx2 = targets_ref[0, :, 2:3]
    ty2 = targets_ref[0, :, 3:4]
    tlab = targets_ref[0, :, 4:5]

    iw = jnp.clip(jnp.minimum(tx2, px2) - jnp.maximum(tx1, px1), 0.0, None)
    ih = jnp.clip(jnp.minimum(ty2, py2) - jnp.maximum(ty1, py1), 0.0, None)
    inter = iw * ih
    area_t = (tx2 - tx1) * (ty2 - ty1)
    area_p = (px2 - px1) * (py2 - py1)
    ov = inter / (area_t + area_p - inter)

    iota_t = lax.broadcasted_iota(jnp.int32, (n_obj, P), 0)
    iota_p = lax.broadcasted_iota(jnp.int32, (n_obj, P), 1)

    bto = jnp.max(ov, axis=0, keepdims=True)
    bti = jnp.min(jnp.where(ov == bto, iota_t, big), axis=0, keepdims=True)
    bpo = jnp.max(ov, axis=1, keepdims=True)
    bpi = jnp.min(jnp.where(ov == bpo, iota_p, big), axis=1, keepdims=True)

    forced_t = jnp.max(jnp.where(iota_p == bpi, iota_t, -1),
                       axis=0, keepdims=True)
    forced = forced_t >= 0
    bto = jnp.where(forced, 2.0, bto)
    bti = jnp.where(forced, forced_t, bti)

    eq = (bti == iota_t).astype(jnp.float32)
    mx1 = jnp.sum(eq * tx1, axis=0, keepdims=True)
    my1 = jnp.sum(eq * ty1, axis=0, keepdims=True)
    mx2 = jnp.sum(eq * tx2, axis=0, keepdims=True)
    my2 = jnp.sum(eq * ty2, axis=0, keepdims=True)
    mlab = jnp.sum(eq * tlab, axis=0, keepdims=True)

    pos = bto >= 0.5
    ign = jnp.logical_and(bto >= 0.4, bto < 0.5)
    posf = pos.astype(jnp.float32)
    ignf = ign.astype(jnp.float32)
    cls = jnp.maximum(mlab.astype(jnp.int32), 0)

    ip = lax.broadcasted_iota(jnp.int32, (1, P), 1)
    gidx_ref[0] = (img * P + ip) * C + cls
    posf_ref[0] = posf
    ignf_ref[0] = ignf

    gcx = ((mx1 + mx2) / 2.0 - pcx) / (VAR0 * pw)
    gcy = ((my1 + my2) / 2.0 - pcy) / (VAR0 * ph)
    gw = jnp.log((mx2 - mx1) / pw) / VAR1
    gh = jnp.log((my2 - my1) / ph) / VAR1
    loc_sum = jnp.zeros((1, 1), jnp.float32)
    for c, g in enumerate((gcx, gcy, gw, gh)):
        d = jnp.abs(loc_ref[0, c:c + 1, :] - g)
        small = (ALPHA_R / _B * (_B * d + 1.0)
                 * jnp.log(_B * d / BETA_R + 1.0) - ALPHA_R * d)
        large = GAMMA_R * d + GAMMA_R / _B - ALPHA_R * BETA_R
        bl = jnp.where(d < BETA_R, small, large)
        loc_sum += jnp.sum(bl * posf, axis=1, keepdims=True)
    loc_sum_ref[0] = loc_sum
    pos_cnt_ref[0] = jnp.sum(posf, axis=1, keepdims=True)
    ign_cnt_ref[0] = jnp.sum(ignf, axis=1, keepdims=True)


def _f0_terms(x):
    u = jnp.exp(-jnp.abs(x))
    sp = jnp.maximum(x, 0.0) + jnp.log1p(u)
    r = 1.0 / (1.0 + u)
    sig = jnp.where(x >= 0.0, r, u * r)
    return sp, sig


def _bulk_kernel(conf_ref, out_ref):
    j = pl.program_id(0)

    @pl.when(j == 0)
    def _init():
        out_ref[...] = jnp.zeros_like(out_ref)

    sp, sig = _f0_terms(conf_ref[...])
    out_ref[...] += (1.0 - ALPHA_F) * jnp.sum(sp * sig, axis=(0, 1),
                                              keepdims=True)


def _corr_kernel(xg_ref, posf_ref, ignf_ref, out_ref):
    x = xg_ref[...]
    sp, sig = _f0_terms(x)
    f0 = (1.0 - ALPHA_F) * sp * sig
    f1 = ALPHA_F * (sp - x) * (1.0 - sig)
    posf = posf_ref[...]
    ignf = ignf_ref[...]
    corr = posf * (f1 - f0) - ignf * f0
    out_ref[...] = jnp.sum(corr, axis=(0, 1), keepdims=True)


def _sc_gather(conf_flat, gidx_flat):
    info = plsc.get_sparse_core_info()
    nw = info.num_cores * info.num_subcores
    b = gidx_flat.shape[0]
    b_per_w = b // nw
    mesh = plsc.VectorSubcoreMesh(core_axis_name="c", subcore_axis_name="s")

    @functools.partial(
        pl.kernel, mesh=mesh,
        out_type=jax.ShapeDtypeStruct((b,), jnp.float32),
        scratch_types=[
            pltpu.VMEM((b_per_w,), jnp.int32),
            pltpu.VMEM((b_per_w,), jnp.float32),
            pltpu.SemaphoreType.DMA,
        ],
    )
    def gather_k(table_hbm, idx_hbm, out_hbm, idx_v, rows_v, sem):
        wid = lax.axis_index("s") * info.num_cores + lax.axis_index("c")
        base = wid * b_per_w
        pltpu.sync_copy(idx_hbm.at[pl.ds(base, b_per_w)], idx_v)
        pltpu.async_copy(table_hbm.at[idx_v], rows_v, sem).wait()
        pltpu.sync_copy(rows_v, out_hbm.at[pl.ds(base, b_per_w)])

    return gather_k(conf_flat, gidx_flat)


@jax.jit
def kernel(loc_data, conf_data, priors, targets):
    num, num_priors, num_classes = conf_data.shape
    P = num_priors

    priors_t = jnp.transpose(priors, (1, 0))
    loc_t = jnp.transpose(loc_data, (0, 2, 1))

    vec_sd = jax.ShapeDtypeStruct((num, 1, P), jnp.int32)
    vecf_sd = jax.ShapeDtypeStruct((num, 1, P), jnp.float32)
    sc_sd = jax.ShapeDtypeStruct((num, 1, 1), jnp.float32)
    gidx, posf, ignf, loc_sum, pos_cnt, ign_cnt = pl.pallas_call(
        _match_kernel,
        grid=(num,),
        in_specs=[
            pl.BlockSpec((4, P), lambda i: (0, 0)),
            pl.BlockSpec((1, targets.shape[1], 5), lambda i: (i, 0, 0)),
            pl.BlockSpec((1, 4, P), lambda i: (i, 0, 0)),
        ],
        out_specs=[pl.BlockSpec((1, 1, P), lambda i: (i, 0, 0))] * 3
        + [pl.BlockSpec((1, 1, 1), lambda i: (i, 0, 0))] * 3,
        out_shape=[vec_sd, vecf_sd, vecf_sd, sc_sd, sc_sd, sc_sd],
        compiler_params=pltpu.CompilerParams(
            dimension_semantics=("arbitrary",),
        ),
    )(priors_t, targets, loc_t)

    total = num * P * num_classes
    conf_flat2 = conf_data.reshape(total // 1280, 1280)
    rows = conf_flat2.shape[0]
    blk = 472
    s0 = pl.pallas_call(
        _bulk_kernel,
        grid=(rows // blk,),
        in_specs=[pl.BlockSpec((blk, 1280), lambda j: (j, 0))],
        out_specs=pl.BlockSpec((1, 1), lambda j: (0, 0)),
        out_shape=jax.ShapeDtypeStruct((1, 1), jnp.float32),
        compiler_params=pltpu.CompilerParams(
            dimension_semantics=("arbitrary",),
        ),
    )(conf_flat2)

    npad = 32 * 128
    bpad = num * P + (-(num * P)) % npad
    gidx_flat = jnp.pad(gidx.reshape(-1), (0, bpad - num * P))
    xg = _sc_gather(conf_data.reshape(-1), gidx_flat)

    posf_flat = jnp.pad(posf.reshape(-1), (0, bpad - num * P))
    ignf_flat = jnp.pad(ignf.reshape(-1), (0, bpad - num * P))
    r2 = bpad // 128
    corr = pl.pallas_call(
        _corr_kernel,
        in_specs=[pl.BlockSpec((r2, 128), lambda: (0, 0))] * 3,
        out_specs=pl.BlockSpec((1, 1), lambda: (0, 0)),
        out_shape=jax.ShapeDtypeStruct((1, 1), jnp.float32),
    )(xg.reshape(r2, 128), posf_flat.reshape(r2, 128),
      ignf_flat.reshape(r2, 128))

    loss_l = jnp.sum(loc_sum) / (4.0 * jnp.sum(pos_cnt))
    denom = jnp.float32(total) - jnp.sum(ign_cnt)
    loss_c = (s0[0, 0] + corr[0, 0]) / denom
    return (loss_l, loss_c)
